# baseline (speedup 1.0000x reference)
import functools

import jax
import jax.numpy as jnp
from jax import lax
from jax.experimental import pallas as pl
from jax.experimental.pallas import tpu as pltpu
from jax.experimental.pallas import tpu_sc as plsc

N_NODES = 10000
N_EDGES = 320000
D = 128
EPS = 1e-5

NC = 2
NS = 16
NW = NC * NS
CH = 80
EPT = 10240
CPT = EPT // CH
GRP = CPT // 8
GW = 8 * CH
E_PAD = NW * EPT
WR = 80
NPAD = 10240
RPT = NPAD // NS
ZR = 128
DEGW = 128

_mesh = plsc.VectorSubcoreMesh(core_axis_name="c", subcore_axis_name="s")


def _agg_body(feat, src, dst, out, acc, ws0, wd0, ws1, wd1,
              rb0, rb1, rb2, rb3, sg0, sg1, sg2, sg3,
              ss0, ss1, ss2, ss3, semsi, semdi):
    c = lax.axis_index("c")
    s = lax.axis_index("s")
    wid = c * NS + s

    rows = [rb0, rb1, rb2, rb3]
    sg = [sg0, sg1, sg2, sg3]
    ss = [ss0, ss1, ss2, ss3]

    @pl.loop(0, WR)
    def _(i):
        for j in range(D // 16):
            rb0[i, pl.ds(j * 16, 16)] = jnp.zeros((16,), jnp.float32)

    for k in range(RPT // WR):
        pltpu.sync_copy(rb0.at[pl.ds(0, WR)],
                        acc.at[pl.ds(s * RPT + k * WR, WR)])
    plsc.subcore_barrier()

    def start_g(slot, wref, roff):
        pltpu.async_copy(feat.at[wref.at[pl.ds(roff * CH, CH)]],
                         rows[slot], sg[slot])

    def drain_g(slot):
        pltpu.make_async_copy(feat.at[ws0.at[pl.ds(0, CH)]],
                              rows[slot], sg[slot]).wait()

    def start_s(slot, wref, roff):
        pltpu.async_copy(rows[slot], acc.at[wref.at[pl.ds(roff * CH, CH)]],
                         ss[slot], add=True)

    def drain_s(slot):
        pltpu.make_async_copy(rows[slot],
                              acc.at[wd0.at[pl.ds(0, CH)]], ss[slot]).wait()

    def group(g, cs, cd, ns, nd, first=False, last=False):
        if not last:
            base = wid * EPT + (g + 1) * GW
            pltpu.async_copy(src.at[pl.ds(base, GW)], ns, semsi)
            pltpu.async_copy(dst.at[pl.ds(base, GW)], nd, semdi)
        for r in range(8):
            s3 = (r + 3) % 4
            sl = r % 4
            if not (first and r == 0):
                drain_s(s3)
            if not (last and r >= 5):
                if r <= 4:
                    start_g(s3, cs, r + 3)
                else:
                    start_g(s3, ns, r - 5)
            if r == 4 and not last:
                pltpu.make_async_copy(src.at[pl.ds(0, GW)], ns, semsi).wait()
                pltpu.make_async_copy(dst.at[pl.ds(0, GW)], nd, semdi).wait()
            drain_g(sl)
            start_s(sl, cd, r)

    pltpu.sync_copy(src.at[pl.ds(wid * EPT, GW)], ws0)
    pltpu.sync_copy(dst.at[pl.ds(wid * EPT, GW)], wd0)
    start_g(0, ws0, 0)
    start_g(1, ws0, 1)
    start_g(2, ws0, 2)

    group(0, ws0, wd0, ws1, wd1, first=True)

    @pl.loop(1, GRP - 1, step=2)
    def _(g):
        group(g, ws1, wd1, ws0, wd0)
        group(g + 1, ws0, wd0, ws1, wd1)

    group(GRP - 1, ws1, wd1, ws0, wd0, last=True)
    drain_s(3)

    plsc.subcore_barrier()

    for k in range(RPT // WR):
        r0 = s * RPT + k * WR
        pltpu.sync_copy(acc.at[pl.ds(r0, WR)], rb0.at[pl.ds(0, WR)])
        pltpu.sync_copy(rb0.at[pl.ds(0, WR)], out.at[c, pl.ds(r0, WR)])


_sc_agg = pl.kernel(
    _agg_body,
    out_type=jax.ShapeDtypeStruct((NC, NPAD, D), jnp.float32),
    mesh=_mesh,
    scratch_types=[
        pltpu.VMEM_SHARED((NPAD, D), jnp.float32),
        pltpu.VMEM((GW,), jnp.int32),
        pltpu.VMEM((GW,), jnp.int32),
        pltpu.VMEM((GW,), jnp.int32),
        pltpu.VMEM((GW,), jnp.int32),
        pltpu.VMEM((CH, D), jnp.float32),
        pltpu.VMEM((CH, D), jnp.float32),
        pltpu.VMEM((CH, D), jnp.float32),
        pltpu.VMEM((CH, D), jnp.float32),
        pltpu.SemaphoreType.DMA,
        pltpu.SemaphoreType.DMA,
        pltpu.SemaphoreType.DMA,
        pltpu.SemaphoreType.DMA,
        pltpu.SemaphoreType.DMA,
        pltpu.SemaphoreType.DMA,
        pltpu.SemaphoreType.DMA,
        pltpu.SemaphoreType.DMA,
        pltpu.SemaphoreType.DMA,
        pltpu.SemaphoreType.DMA,
    ],
)


def _deg_body(dst, out, acc, dstv, ones, zbuf):
    c = lax.axis_index("c")
    s = lax.axis_index("s")
    wid = c * NS + s

    pltpu.sync_copy(dst.at[pl.ds(wid * EPT, EPT)], dstv)

    @pl.loop(0, CH)
    def _(i):
        for j in range(DEGW // 16):
            ones[i, pl.ds(j * 16, 16)] = jnp.ones((16,), jnp.float32)

    @pl.loop(0, WR)
    def _(i):
        for j in range(DEGW // 16):
            zbuf[i, pl.ds(j * 16, 16)] = jnp.zeros((16,), jnp.float32)

    for k in range(RPT // WR):
        pltpu.sync_copy(zbuf.at[pl.ds(0, WR)],
                        acc.at[pl.ds(s * RPT + k * WR, WR)])
    plsc.subcore_barrier()

    @pl.loop(0, CPT)
    def _(j):
        pltpu.sync_copy(ones, acc.at[dstv.at[pl.ds(j * CH, CH)]], add=True)

    plsc.subcore_barrier()

    for k in range(RPT // WR):
        r0 = s * RPT + k * WR
        pltpu.sync_copy(acc.at[pl.ds(r0, WR)], zbuf.at[pl.ds(0, WR)])
        pltpu.sync_copy(zbuf.at[pl.ds(0, WR)], out.at[c, pl.ds(r0, WR)])


_sc_deg = pl.kernel(
    _deg_body,
    out_type=jax.ShapeDtypeStruct((NC, NPAD, DEGW), jnp.float32),
    mesh=_mesh,
    scratch_types=[
        pltpu.VMEM_SHARED((NPAD, DEGW), jnp.float32),
        pltpu.VMEM((EPT,), jnp.int32),
        pltpu.VMEM((CH, DEGW), jnp.float32),
        pltpu.VMEM((CH, DEGW), jnp.float32),
    ],
)


def _proj_body(x_ref, wst_ref, wnt_ref, ps_ref, pn_ref):
    h = x_ref[...]
    ps_ref[...] = jnp.dot(h, wst_ref[...], preferred_element_type=jnp.float32)
    pn_ref[...] = jnp.dot(h, wnt_ref[...], preferred_element_type=jnp.float32)


_tc_proj = pl.pallas_call(
    _proj_body,
    out_shape=[
        jax.ShapeDtypeStruct((N_NODES, D), jnp.float32),
        jax.ShapeDtypeStruct((N_NODES, D), jnp.float32),
    ],
)


def _combine(ps_ref, sp_ref, degp_ref):
    sp = sp_ref[...]
    degp = degp_ref[...]
    deg = degp[0, :N_NODES] + degp[1, :N_NODES]
    invd = 1.0 / jnp.maximum(deg[:, 0:1], 1.0)
    return ps_ref[...] + (sp[0, :N_NODES] + sp[1, :N_NODES]) * invd


def _mid_body(ps_ref, sp_ref, degp_ref, g_ref, b_ref, wst_ref, wnt_ref,
              ps_out, pn_out):
    z = _combine(ps_ref, sp_ref, degp_ref)
    mean = jnp.mean(z, axis=0, keepdims=True)
    var = jnp.mean((z - mean) ** 2, axis=0, keepdims=True)
    h = (z - mean) * lax.rsqrt(var + EPS) * g_ref[...] + b_ref[...]
    h = jnp.maximum(h, 0.0)
    ps_out[...] = jnp.dot(h, wst_ref[...], preferred_element_type=jnp.float32)
    pn_out[...] = jnp.dot(h, wnt_ref[...], preferred_element_type=jnp.float32)


_tc_mid = pl.pallas_call(
    _mid_body,
    out_shape=[
        jax.ShapeDtypeStruct((N_NODES, D), jnp.float32),
        jax.ShapeDtypeStruct((N_NODES, D), jnp.float32),
    ],
)


def _final_body(ps_ref, sp_ref, degp_ref, o_ref):
    z = _combine(ps_ref, sp_ref, degp_ref)
    m = jnp.max(z, axis=-1, keepdims=True)
    lse = jnp.log(jnp.sum(jnp.exp(z - m), axis=-1, keepdims=True)) + m
    o_ref[...] = z - lse


_tc_final = pl.pallas_call(
    _final_body,
    out_shape=jax.ShapeDtypeStruct((N_NODES, D), jnp.float32),
)


def kernel(x, edge_index, W_self0, W_neigh0, W_self1, W_neigh1,
           W_self2, W_neigh2, gamma0, beta0, gamma1, beta1):
    ei = edge_index.astype(jnp.int32)
    npad_e = E_PAD - N_EDGES
    src1d = jnp.concatenate([ei[0], jnp.zeros((npad_e,), jnp.int32)])
    pad_rows = N_NODES + (jnp.arange(npad_e, dtype=jnp.int32) % (NPAD - N_NODES))
    dst1d = jnp.concatenate([ei[1], pad_rows])
    g0 = gamma0.reshape(1, D)
    b0 = beta0.reshape(1, D)
    g1 = gamma1.reshape(1, D)
    b1 = beta1.reshape(1, D)

    degp = _sc_deg(dst1d)
    ps0, pn0 = _tc_proj(x, W_self0.T, W_neigh0.T)
    sp0 = _sc_agg(pn0, src1d, dst1d)
    ps1, pn1 = _tc_mid(ps0, sp0, degp, g0, b0, W_self1.T, W_neigh1.T)
    sp1 = _sc_agg(pn1, src1d, dst1d)
    ps2, pn2 = _tc_mid(ps1, sp1, degp, g1, b1, W_self2.T, W_neigh2.T)
    sp2 = _sc_agg(pn2, src1d, dst1d)
    return _tc_final(ps2, sp2, degp)


# speedup vs baseline: 3.3923x; 3.3923x over previous
import functools

import jax
import jax.numpy as jnp
from jax import lax
from jax.experimental import pallas as pl
from jax.experimental.pallas import tpu as pltpu
from jax.experimental.pallas import tpu_sc as plsc

N_NODES = 10000
N_EDGES = 320000
D = 128
EPS = 1e-5

NC = 2
NS = 16
NW = NC * NS
CH = 80
EPT = 10000
CPT = EPT // CH
E_PAD = NW * EPT
WR = 80
NPAD = 10240
RPT = NPAD // NS
ZR = 128
DEGW = 128

_mesh = plsc.VectorSubcoreMesh(core_axis_name="c", subcore_axis_name="s")


def _agg_body(feat, src, dst, out, acc, srcv, dstv, rows_a, rows_b,
              sem_a, sem_b):
    c = lax.axis_index("c")
    s = lax.axis_index("s")
    wid = c * NS + s

    pltpu.sync_copy(src.at[pl.ds(wid * EPT, EPT)], srcv)
    pltpu.sync_copy(dst.at[pl.ds(wid * EPT, EPT)], dstv)

    @pl.loop(0, WR)
    def _(i):
        for j in range(D // 16):
            rows_a[i, pl.ds(j * 16, 16)] = jnp.zeros((16,), jnp.float32)

    for k in range(RPT // WR):
        pltpu.sync_copy(rows_a.at[pl.ds(0, WR)],
                        acc.at[pl.ds(s * RPT + k * WR, WR)])
    plsc.subcore_barrier()

    def start(buf, sem, j):
        pltpu.async_copy(feat.at[srcv.at[pl.ds(j * CH, CH)]], buf, sem)

    def drain(buf, sem):
        pltpu.make_async_copy(feat.at[srcv.at[pl.ds(0, CH)]], buf, sem).wait()

    def scat(buf, j):
        pltpu.sync_copy(buf, acc.at[dstv.at[pl.ds(j * CH, CH)]], add=True)

    start(rows_a, sem_a, 0)

    @pl.loop(0, CPT - 1, step=2)
    def _(j):
        start(rows_b, sem_b, j + 1)
        drain(rows_a, sem_a)
        scat(rows_a, j)
        start(rows_a, sem_a, j + 2)
        drain(rows_b, sem_b)
        scat(rows_b, j + 1)

    drain(rows_a, sem_a)
    scat(rows_a, CPT - 1)

    plsc.subcore_barrier()

    for k in range(RPT // WR):
        r0 = s * RPT + k * WR
        pltpu.sync_copy(acc.at[pl.ds(r0, WR)], rows_a.at[pl.ds(0, WR)])
        pltpu.sync_copy(rows_a.at[pl.ds(0, WR)], out.at[c, pl.ds(r0, WR)])


_sc_agg = pl.kernel(
    _agg_body,
    out_type=jax.ShapeDtypeStruct((NC, NPAD, D), jnp.float32),
    mesh=_mesh,
    scratch_types=[
        pltpu.VMEM_SHARED((NPAD, D), jnp.float32),
        pltpu.VMEM((EPT,), jnp.int32),
        pltpu.VMEM((EPT,), jnp.int32),
        pltpu.VMEM((CH, D), jnp.float32),
        pltpu.VMEM((CH, D), jnp.float32),
        pltpu.SemaphoreType.DMA,
        pltpu.SemaphoreType.DMA,
    ],
)


def _deg_body(dst, out, acc, dstv, ones, zbuf):
    c = lax.axis_index("c")
    s = lax.axis_index("s")
    wid = c * NS + s

    pltpu.sync_copy(dst.at[pl.ds(wid * EPT, EPT)], dstv)

    @pl.loop(0, CH)
    def _(i):
        for j in range(DEGW // 16):
            ones[i, pl.ds(j * 16, 16)] = jnp.ones((16,), jnp.float32)

    @pl.loop(0, WR)
    def _(i):
        for j in range(DEGW // 16):
            zbuf[i, pl.ds(j * 16, 16)] = jnp.zeros((16,), jnp.float32)

    for k in range(RPT // WR):
        pltpu.sync_copy(zbuf.at[pl.ds(0, WR)],
                        acc.at[pl.ds(s * RPT + k * WR, WR)])
    plsc.subcore_barrier()

    @pl.loop(0, CPT)
    def _(j):
        pltpu.sync_copy(ones, acc.at[dstv.at[pl.ds(j * CH, CH)]], add=True)

    plsc.subcore_barrier()

    for k in range(RPT // WR):
        r0 = s * RPT + k * WR
        pltpu.sync_copy(acc.at[pl.ds(r0, WR)], zbuf.at[pl.ds(0, WR)])
        pltpu.sync_copy(zbuf.at[pl.ds(0, WR)], out.at[c, pl.ds(r0, WR)])


_sc_deg = pl.kernel(
    _deg_body,
    out_type=jax.ShapeDtypeStruct((NC, NPAD, DEGW), jnp.float32),
    mesh=_mesh,
    scratch_types=[
        pltpu.VMEM_SHARED((NPAD, DEGW), jnp.float32),
        pltpu.VMEM((EPT,), jnp.int32),
        pltpu.VMEM((CH, DEGW), jnp.float32),
        pltpu.VMEM((CH, DEGW), jnp.float32),
    ],
)


def _proj_body(x_ref, wst_ref, wnt_ref, ps_ref, pn_ref):
    h = x_ref[...]
    ps_ref[...] = jnp.dot(h, wst_ref[...], preferred_element_type=jnp.float32)
    pn_ref[...] = jnp.dot(h, wnt_ref[...], preferred_element_type=jnp.float32)


_tc_proj = pl.pallas_call(
    _proj_body,
    out_shape=[
        jax.ShapeDtypeStruct((N_NODES, D), jnp.float32),
        jax.ShapeDtypeStruct((N_NODES, D), jnp.float32),
    ],
)


def _combine(ps_ref, sp_ref, degp_ref):
    sp = sp_ref[...]
    degp = degp_ref[...]
    deg = degp[0, :N_NODES] + degp[1, :N_NODES]
    invd = 1.0 / jnp.maximum(deg[:, 0:1], 1.0)
    return ps_ref[...] + (sp[0, :N_NODES] + sp[1, :N_NODES]) * invd


def _mid_body(ps_ref, sp_ref, degp_ref, g_ref, b_ref, wst_ref, wnt_ref,
              ps_out, pn_out):
    z = _combine(ps_ref, sp_ref, degp_ref)
    mean = jnp.mean(z, axis=0, keepdims=True)
    var = jnp.mean((z - mean) ** 2, axis=0, keepdims=True)
    h = (z - mean) * lax.rsqrt(var + EPS) * g_ref[...] + b_ref[...]
    h = jnp.maximum(h, 0.0)
    ps_out[...] = jnp.dot(h, wst_ref[...], preferred_element_type=jnp.float32)
    pn_out[...] = jnp.dot(h, wnt_ref[...], preferred_element_type=jnp.float32)


_tc_mid = pl.pallas_call(
    _mid_body,
    out_shape=[
        jax.ShapeDtypeStruct((N_NODES, D), jnp.float32),
        jax.ShapeDtypeStruct((N_NODES, D), jnp.float32),
    ],
)


def _final_body(ps_ref, sp_ref, degp_ref, o_ref):
    z = _combine(ps_ref, sp_ref, degp_ref)
    m = jnp.max(z, axis=-1, keepdims=True)
    lse = jnp.log(jnp.sum(jnp.exp(z - m), axis=-1, keepdims=True)) + m
    o_ref[...] = z - lse


_tc_final = pl.pallas_call(
    _final_body,
    out_shape=jax.ShapeDtypeStruct((N_NODES, D), jnp.float32),
)


def kernel(x, edge_index, W_self0, W_neigh0, W_self1, W_neigh1,
           W_self2, W_neigh2, gamma0, beta0, gamma1, beta1):
    ei = edge_index.astype(jnp.int32)
    npad_e = E_PAD - N_EDGES
    src1d = jnp.concatenate([ei[0], jnp.zeros((npad_e,), jnp.int32)])
    pad_rows = N_NODES + (jnp.arange(npad_e, dtype=jnp.int32) % (NPAD - N_NODES))
    dst1d = jnp.concatenate([ei[1], pad_rows])
    g0 = gamma0.reshape(1, D)
    b0 = beta0.reshape(1, D)
    g1 = gamma1.reshape(1, D)
    b1 = beta1.reshape(1, D)

    degp = _sc_deg(dst1d)
    ps0, pn0 = _tc_proj(x, W_self0.T, W_neigh0.T)
    sp0 = _sc_agg(pn0, src1d, dst1d)
    ps1, pn1 = _tc_mid(ps0, sp0, degp, g0, b0, W_self1.T, W_neigh1.T)
    sp1 = _sc_agg(pn1, src1d, dst1d)
    ps2, pn2 = _tc_mid(ps1, sp1, degp, g1, b1, W_self2.T, W_neigh2.T)
    sp2 = _sc_agg(pn2, src1d, dst1d)
    return _tc_final(ps2, sp2, degp)
